# baseline (speedup 1.0000x reference)
import functools

import jax
import jax.numpy as jnp
from jax.experimental import pallas as pl
from jax.experimental.pallas import tpu as pltpu

_B = 256
_C = 256


def _metav_body(E, tki_ref, loc_ref, counts_ref, starts8_ref):
    NC = tki_ref.shape[0]
    sub_e = jax.lax.broadcasted_iota(jnp.int32, (E, _C), 0)
    row_i = jax.lax.broadcasted_iota(jnp.int32, (_C, _C), 0)
    col_i = jax.lax.broadcasted_iota(jnp.int32, (_C, _C), 1)
    utri = (row_i < col_i).astype(jnp.bfloat16)

    def count_chunk(c, acc):
        ech = tki_ref[pl.ds(c, 1), :]
        oh = (ech == sub_e).astype(jnp.float32)
        return acc + jnp.sum(oh, axis=1, keepdims=True)

    counts_f = jax.lax.fori_loop(
        0, NC, count_chunk, jnp.zeros((E, 1), jnp.float32))
    counts = counts_f.astype(jnp.int32)
    counts8 = ((counts + 7) >> 3) << 3
    sub_idx = jax.lax.broadcasted_iota(jnp.int32, (E, 1), 0)
    acc = counts8
    sh = 1
    while sh < E:
        r = pltpu.roll(acc, sh, 0)
        acc = acc + jnp.where(sub_idx >= sh, r, 0)
        sh *= 2
    starts8 = acc - counts8
    counts_ref[...] = counts
    starts8_ref[...] = starts8
    base_f = starts8.astype(jnp.float32)

    def place_chunk(c, cum):
        ech = tki_ref[pl.ds(c, 1), :]
        ohf = (ech == sub_e).astype(jnp.float32)
        rank = jnp.dot(ohf.astype(jnp.bfloat16), utri,
                       preferred_element_type=jnp.float32)
        pos = jnp.sum(ohf * (base_f + cum + rank), axis=0, keepdims=True)
        loc_ref[pl.ds(c, 1), :] = pos.astype(jnp.int32)
        return cum + jnp.sum(ohf, axis=1, keepdims=True)

    jax.lax.fori_loop(0, NC, place_chunk, jnp.zeros((E, 1), jnp.float32))


def _gemm_body(T, K,
               st_ref, cnt_ref, loc_ref, tkw_ref,
               hs_ref, wup_ref, wdn_ref,
               out_ref,
               x_ref, ys_ref, rows_ref):
    e = pl.program_id(0)
    num_e = pl.num_programs(0)
    P = loc_ref.shape[0]

    @pl.when(e == 0)
    def _():
        def place(p, c):
            rows_ref[loc_ref[p]] = p // K
            return c

        jax.lax.fori_loop(0, P, place, jnp.int32(0))

    cnt = cnt_ref[e]
    start = st_ref[e]
    nblk = (cnt + _B - 1) // _B

    def blk(j, carry):
        base = pl.multiple_of(start + j * _B, 8)
        for i in range(_B):
            r = jnp.clip(rows_ref[base + i], 0, T - 1)
            x_ref[i, :] = hs_ref[r, :]
        xb = x_ref[...].astype(jnp.bfloat16)
        h = jnp.dot(xb, wup_ref[0].astype(jnp.bfloat16),
                    preferred_element_type=jnp.float32)
        h = jnp.maximum(h, 0.0).astype(jnp.bfloat16)
        y = jnp.dot(h, wdn_ref[0].astype(jnp.bfloat16),
                    preferred_element_type=jnp.float32)
        ys_ref[pl.ds(base, _B), :] = y
        return carry

    jax.lax.fori_loop(0, nblk, blk, 0)

    @pl.when(e == num_e - 1)
    def _():
        def group(g, carry):
            for i in range(_B):
                t = g * _B + i
                acc = ys_ref[loc_ref[K * t], :] * tkw_ref[K * t]
                for k in range(1, K):
                    acc = acc + ys_ref[loc_ref[K * t + k], :] * tkw_ref[K * t + k]
                x_ref[i, :] = acc
            out_ref[pl.ds(pl.multiple_of(g * _B, 8), _B), :] = x_ref[...]
            return carry

        jax.lax.fori_loop(0, T // _B, group, jnp.int32(0))


def kernel(hidden_states, top_k_index, top_k_weights, W_up, W_down):
    T, D = hidden_states.shape
    _, K = top_k_index.shape
    E = W_up.shape[0]
    P = T * K
    P8 = P + 8 * E
    NC = P // _C

    metav = pl.pallas_call(
        functools.partial(_metav_body, E),
        out_shape=[
            jax.ShapeDtypeStruct((NC, _C), jnp.int32),
            jax.ShapeDtypeStruct((E, 1), jnp.int32),
            jax.ShapeDtypeStruct((E, 1), jnp.int32),
        ],
    )
    loc_col, counts_row, starts8_row = metav(top_k_index.reshape(NC, _C))
    loc = loc_col.reshape(P)
    counts = counts_row.reshape(E)
    starts8 = starts8_row.reshape(E)

    gemm = pl.pallas_call(
        functools.partial(_gemm_body, T, K),
        grid_spec=pltpu.PrefetchScalarGridSpec(
            num_scalar_prefetch=4,
            grid=(E,),
            in_specs=[
                pl.BlockSpec((T, D), lambda e, *_: (0, 0)),
                pl.BlockSpec((1, D, D), lambda e, *_: (e, 0, 0)),
                pl.BlockSpec((1, D, D), lambda e, *_: (e, 0, 0)),
            ],
            out_specs=pl.BlockSpec((T, D), lambda e, *_: (0, 0)),
            scratch_shapes=[
                pltpu.VMEM((_B, D), jnp.float32),
                pltpu.VMEM((P8 + _B, D), jnp.float32),
                pltpu.SMEM((P8 + _B,), jnp.int32),
            ],
        ),
        out_shape=jax.ShapeDtypeStruct((T, D), jnp.float32),
        compiler_params=pltpu.CompilerParams(
            dimension_semantics=("arbitrary",)),
    )
    out = gemm(starts8, counts, loc, top_k_weights.reshape(P),
               hidden_states, W_up, W_down)
    return out.astype(hidden_states.dtype)


# speedup vs baseline: 1.1138x; 1.1138x over previous
import functools

import jax
import jax.numpy as jnp
from jax.experimental import pallas as pl
from jax.experimental.pallas import tpu as pltpu

_B = 128
_C = 256


def _metav_body(E, tki_ref, loc_ref, counts_ref, starts8_ref):
    NC = tki_ref.shape[0]
    sub_e = jax.lax.broadcasted_iota(jnp.int32, (E, _C), 0)
    row_i = jax.lax.broadcasted_iota(jnp.int32, (_C, _C), 0)
    col_i = jax.lax.broadcasted_iota(jnp.int32, (_C, _C), 1)
    utri = (row_i < col_i).astype(jnp.bfloat16)

    def count_chunk(c, acc):
        ech = tki_ref[pl.ds(c, 1), :]
        oh = (ech == sub_e).astype(jnp.float32)
        return acc + jnp.sum(oh, axis=1, keepdims=True)

    counts_f = jax.lax.fori_loop(
        0, NC, count_chunk, jnp.zeros((E, 1), jnp.float32))
    counts = counts_f.astype(jnp.int32)
    counts8 = ((counts + 7) >> 3) << 3
    sub_idx = jax.lax.broadcasted_iota(jnp.int32, (E, 1), 0)
    acc = counts8
    sh = 1
    while sh < E:
        r = pltpu.roll(acc, sh, 0)
        acc = acc + jnp.where(sub_idx >= sh, r, 0)
        sh *= 2
    starts8 = acc - counts8
    counts_ref[...] = counts
    starts8_ref[...] = starts8
    base_f = starts8.astype(jnp.float32)

    def place_chunk(c, cum):
        ech = tki_ref[pl.ds(c, 1), :]
        ohf = (ech == sub_e).astype(jnp.float32)
        rank = jnp.dot(ohf.astype(jnp.bfloat16), utri,
                       preferred_element_type=jnp.float32)
        pos = jnp.sum(ohf * (base_f + cum + rank), axis=0, keepdims=True)
        loc_ref[pl.ds(c, 1), :] = pos.astype(jnp.int32)
        return cum + jnp.sum(ohf, axis=1, keepdims=True)

    jax.lax.fori_loop(0, NC, place_chunk, jnp.zeros((E, 1), jnp.float32))


def _gemm_body(T, K,
               st_ref, cnt_ref, loc_ref, tkw_ref,
               hs_ref, wup_ref, wdn_ref,
               out_ref,
               x_ref, ys_ref, rows_ref):
    e = pl.program_id(0)
    num_e = pl.num_programs(0)
    P = loc_ref.shape[0]

    @pl.when(e == 0)
    def _():
        def place(p, c):
            rows_ref[loc_ref[p]] = p // K
            return c

        jax.lax.fori_loop(0, P, place, jnp.int32(0))

    cnt = cnt_ref[e]
    start = st_ref[e]
    nblk = (cnt + _B - 1) // _B

    def blk(j, carry):
        base = pl.multiple_of(start + j * _B, 8)
        for i in range(_B):
            r = jnp.clip(rows_ref[base + i], 0, T - 1)
            x_ref[i, :] = hs_ref[r, :]
        xb = x_ref[...].astype(jnp.bfloat16)
        h = jnp.dot(xb, wup_ref[0].astype(jnp.bfloat16),
                    preferred_element_type=jnp.float32)
        h = jnp.maximum(h, 0.0).astype(jnp.bfloat16)
        y = jnp.dot(h, wdn_ref[0].astype(jnp.bfloat16),
                    preferred_element_type=jnp.float32)
        ys_ref[pl.ds(base, _B), :] = y
        return carry

    jax.lax.fori_loop(0, nblk, blk, 0)

    @pl.when(e == num_e - 1)
    def _():
        def group(g, carry):
            for i in range(_B):
                t = g * _B + i
                acc = ys_ref[loc_ref[K * t], :] * tkw_ref[K * t]
                for k in range(1, K):
                    acc = acc + ys_ref[loc_ref[K * t + k], :] * tkw_ref[K * t + k]
                x_ref[i, :] = acc
            out_ref[pl.ds(pl.multiple_of(g * _B, 8), _B), :] = x_ref[...]
            return carry

        jax.lax.fori_loop(0, T // _B, group, jnp.int32(0))


def kernel(hidden_states, top_k_index, top_k_weights, W_up, W_down):
    T, D = hidden_states.shape
    _, K = top_k_index.shape
    E = W_up.shape[0]
    P = T * K
    P8 = P + 8 * E
    NC = P // _C

    metav = pl.pallas_call(
        functools.partial(_metav_body, E),
        out_shape=[
            jax.ShapeDtypeStruct((NC, _C), jnp.int32),
            jax.ShapeDtypeStruct((E, 1), jnp.int32),
            jax.ShapeDtypeStruct((E, 1), jnp.int32),
        ],
    )
    loc_col, counts_row, starts8_row = metav(top_k_index.reshape(NC, _C))
    loc = loc_col.reshape(P)
    counts = counts_row.reshape(E)
    starts8 = starts8_row.reshape(E)

    gemm = pl.pallas_call(
        functools.partial(_gemm_body, T, K),
        grid_spec=pltpu.PrefetchScalarGridSpec(
            num_scalar_prefetch=4,
            grid=(E,),
            in_specs=[
                pl.BlockSpec((T, D), lambda e, *_: (0, 0)),
                pl.BlockSpec((1, D, D), lambda e, *_: (e, 0, 0)),
                pl.BlockSpec((1, D, D), lambda e, *_: (e, 0, 0)),
            ],
            out_specs=pl.BlockSpec((T, D), lambda e, *_: (0, 0)),
            scratch_shapes=[
                pltpu.VMEM((_B, D), jnp.float32),
                pltpu.VMEM((P8 + _B, D), jnp.float32),
                pltpu.SMEM((P8 + _B,), jnp.int32),
            ],
        ),
        out_shape=jax.ShapeDtypeStruct((T, D), jnp.float32),
        compiler_params=pltpu.CompilerParams(
            dimension_semantics=("arbitrary",)),
    )
    out = gemm(starts8, counts, loc, top_k_weights.reshape(P),
               hidden_states, W_up, W_down)
    return out.astype(hidden_states.dtype)


# speedup vs baseline: 1.1863x; 1.0651x over previous
import functools

import jax
import jax.numpy as jnp
from jax.experimental import pallas as pl
from jax.experimental.pallas import tpu as pltpu

_B = 128
_C = 256


def _metav_body(E, tki_ref, loc_ref, counts_ref, starts8_ref):
    NC = tki_ref.shape[0]
    sub_e = jax.lax.broadcasted_iota(jnp.int32, (E, _C), 0)
    row_i = jax.lax.broadcasted_iota(jnp.int32, (_C, _C), 0)
    col_i = jax.lax.broadcasted_iota(jnp.int32, (_C, _C), 1)
    utri = (row_i < col_i).astype(jnp.bfloat16)

    def count_chunk(c, acc):
        ech = tki_ref[pl.ds(c, 1), :]
        oh = (ech == sub_e).astype(jnp.float32)
        return acc + jnp.sum(oh, axis=1, keepdims=True)

    counts_f = jax.lax.fori_loop(
        0, NC, count_chunk, jnp.zeros((E, 1), jnp.float32))
    counts = counts_f.astype(jnp.int32)
    counts8 = ((counts + 7) >> 3) << 3
    sub_idx = jax.lax.broadcasted_iota(jnp.int32, (E, 1), 0)
    acc = counts8
    sh = 1
    while sh < E:
        r = pltpu.roll(acc, sh, 0)
        acc = acc + jnp.where(sub_idx >= sh, r, 0)
        sh *= 2
    starts8 = acc - counts8
    counts_ref[...] = counts
    starts8_ref[...] = starts8
    base_f = starts8.astype(jnp.float32)

    def place_chunk(c, cum):
        ech = tki_ref[pl.ds(c, 1), :]
        ohf = (ech == sub_e).astype(jnp.float32)
        rank = jnp.dot(ohf.astype(jnp.bfloat16), utri,
                       preferred_element_type=jnp.float32)
        pos = jnp.sum(ohf * (base_f + cum + rank), axis=0, keepdims=True)
        loc_ref[pl.ds(c, 1), :] = pos.astype(jnp.int32)
        return cum + jnp.sum(ohf, axis=1, keepdims=True)

    jax.lax.fori_loop(0, NC, place_chunk, jnp.zeros((E, 1), jnp.float32))


def _gemm_body(T, K,
               st_ref, cnt_ref, loc_ref, tkw_ref,
               hs_ref, wup_ref, wdn_ref,
               out_ref,
               x_ref, ys_ref, rows_ref):
    e = pl.program_id(0)
    num_e = pl.num_programs(0)
    P = loc_ref.shape[0]

    @pl.when(e == 0)
    def _():
        def place(p8, c):
            for u in range(8):
                p = p8 * 8 + u
                rows_ref[loc_ref[p]] = p // K
            return c

        jax.lax.fori_loop(0, P // 8, place, jnp.int32(0))

    cnt = cnt_ref[e]
    start = st_ref[e]
    nblk = (cnt + _B - 1) // _B

    def blk(j, carry):
        base = pl.multiple_of(start + j * _B, 8)
        for i in range(_B):
            r = jnp.clip(rows_ref[base + i], 0, T - 1)
            x_ref[i, :] = hs_ref[r, :]
        xb = x_ref[...].astype(jnp.bfloat16)
        h = jnp.dot(xb, wup_ref[0].astype(jnp.bfloat16),
                    preferred_element_type=jnp.float32)
        h = jnp.maximum(h, 0.0).astype(jnp.bfloat16)
        y = jnp.dot(h, wdn_ref[0].astype(jnp.bfloat16),
                    preferred_element_type=jnp.float32)
        ys_ref[pl.ds(base, _B), :] = y
        return carry

    jax.lax.fori_loop(0, nblk, blk, 0)

    @pl.when(e == num_e - 1)
    def _():
        def group(g, carry):
            for i in range(_B):
                t = g * _B + i
                acc = ys_ref[loc_ref[K * t], :] * tkw_ref[K * t]
                for k in range(1, K):
                    acc = acc + ys_ref[loc_ref[K * t + k], :] * tkw_ref[K * t + k]
                x_ref[i, :] = acc
            out_ref[pl.ds(pl.multiple_of(g * _B, 8), _B), :] = x_ref[...]
            return carry

        jax.lax.fori_loop(0, T // _B, group, jnp.int32(0))


def kernel(hidden_states, top_k_index, top_k_weights, W_up, W_down):
    T, D = hidden_states.shape
    _, K = top_k_index.shape
    E = W_up.shape[0]
    P = T * K
    P8 = P + 8 * E
    NC = P // _C

    metav = pl.pallas_call(
        functools.partial(_metav_body, E),
        out_shape=[
            jax.ShapeDtypeStruct((NC, _C), jnp.int32),
            jax.ShapeDtypeStruct((E, 1), jnp.int32),
            jax.ShapeDtypeStruct((E, 1), jnp.int32),
        ],
    )
    loc_col, counts_row, starts8_row = metav(top_k_index.reshape(NC, _C))
    loc = loc_col.reshape(P)
    counts = counts_row.reshape(E)
    starts8 = starts8_row.reshape(E)

    gemm = pl.pallas_call(
        functools.partial(_gemm_body, T, K),
        grid_spec=pltpu.PrefetchScalarGridSpec(
            num_scalar_prefetch=4,
            grid=(E,),
            in_specs=[
                pl.BlockSpec((T, D), lambda e, *_: (0, 0)),
                pl.BlockSpec((1, D, D), lambda e, *_: (e, 0, 0)),
                pl.BlockSpec((1, D, D), lambda e, *_: (e, 0, 0)),
            ],
            out_specs=pl.BlockSpec((T, D), lambda e, *_: (0, 0)),
            scratch_shapes=[
                pltpu.VMEM((_B, D), jnp.float32),
                pltpu.VMEM((P8 + _B, D), jnp.float32),
                pltpu.SMEM((P8 + _B,), jnp.int32),
            ],
        ),
        out_shape=jax.ShapeDtypeStruct((T, D), jnp.float32),
        compiler_params=pltpu.CompilerParams(
            dimension_semantics=("arbitrary",)),
    )
    out = gemm(starts8, counts, loc, top_k_weights.reshape(P),
               hidden_states, W_up, W_down)
    return out.astype(hidden_states.dtype)
